# Initial kernel scaffold; baseline (speedup 1.0000x reference)
import jax
import jax.numpy as jnp
from jax.experimental import pallas as pl


def kernel(x, edge_index, Wl1, bl1, Wr1, g1, be1, Wl2, bl2, Wr2, g2, be2, Wl4, bl4, Wr4, Wls, bls, Wrs, gs, bes, Wlr, blr, Wrr, gr, ber, Wr3, br3, Wlm, blm, Wrm, gm, bem, Wm3, bm3):
    raise NotImplementedError("write your pallas kernel here")


# speedup vs baseline: 3.0101x; 3.0101x over previous
import functools

import jax
import jax.numpy as jnp
from jax import lax
from jax.experimental import pallas as pl
from jax.experimental.pallas import tpu as pltpu
from jax.experimental.pallas import tpu_sc as plsc

N = 10000
NT = 32
NPT = 320
NPAD = NT * NPT
E = 320000
CHUNK = 2560
G = 128
LCAP = E + G

_MESH = plsc.VectorSubcoreMesh(core_axis_name="c", subcore_axis_name="s")


def _wid():
    return lax.axis_index("s") * 2 + lax.axis_index("c")


@functools.partial(
    pl.kernel,
    out_type=(
        jax.ShapeDtypeStruct((NT * LCAP,), jnp.int32),
        jax.ShapeDtypeStruct((NT * LCAP,), jnp.int32),
        jax.ShapeDtypeStruct((NT * 16,), jnp.int32),
    ),
    mesh=_MESH,
    compiler_params=pltpu.CompilerParams(needs_layout_passes=False),
    scratch_types=[
        pltpu.VMEM((CHUNK,), jnp.int32),
        pltpu.VMEM((CHUNK,), jnp.int32),
        pltpu.VMEM((G + 16,), jnp.int32),
        pltpu.VMEM((G + 16,), jnp.int32),
        pltpu.VMEM((16,), jnp.int32),
    ],
)
def _bin_edges(src_hbm, dst_hbm, srcl_hbm, ldl_hbm, cnt_hbm,
               srcb, dstb, sstage, lstage, cntv):
    wid = _wid()
    lo = wid * NPT
    hi = lo + NPT
    iota16 = lax.iota(jnp.int32, 16)

    def _flush(c):
        pos, off = c
        pltpu.sync_copy(sstage.at[pl.ds(0, G)],
                        srcl_hbm.at[pl.ds(pl.multiple_of(wid * LCAP + off, 8), G)])
        pltpu.sync_copy(lstage.at[pl.ds(0, G)],
                        ldl_hbm.at[pl.ds(pl.multiple_of(wid * LCAP + off, 8), G)])
        sstage[pl.ds(0, 16)] = sstage[pl.ds(G, 16)]
        lstage[pl.ds(0, 16)] = lstage[pl.ds(G, 16)]
        return (pos - G, off + G)

    def chunk_body(k, carry):
        pltpu.sync_copy(src_hbm.at[pl.ds(k * CHUNK, CHUNK)], srcb)
        pltpu.sync_copy(dst_hbm.at[pl.ds(k * CHUNK, CHUNK)], dstb)

        def vreg_body(i, c2):
            pos, off = c2
            dv = dstb[pl.ds(i * 16, 16)]
            sv = srcb[pl.ds(i * 16, 16)]
            m = (dv >= lo) & (dv < hi)
            m32 = m.astype(jnp.int32)
            cs = jnp.cumsum(m32)
            positions = pos + cs - m32
            plsc.store_scatter(sstage, [positions], sv, mask=m)
            plsc.store_scatter(lstage, [positions], dv - lo, mask=m)
            pos = pos + cs[15]
            return lax.cond(pos >= G, _flush, lambda c: c, (pos, off))

        return lax.fori_loop(0, CHUNK // 16, vreg_body, carry)

    pos, off = lax.fori_loop(0, E // CHUNK, chunk_body, (0, 0))

    padsrc = lo + iota16
    padld = jnp.zeros((16,), jnp.int32) + NPT
    for j16 in range(G // 16):
        idxs = j16 * 16 + iota16
        mpad = idxs >= pos
        plsc.store_scatter(sstage, [idxs], padsrc, mask=mpad)
        plsc.store_scatter(lstage, [idxs], padld, mask=mpad)
    pltpu.sync_copy(sstage.at[pl.ds(0, G)],
                    srcl_hbm.at[pl.ds(pl.multiple_of(wid * LCAP + off, 8), G)])
    pltpu.sync_copy(lstage.at[pl.ds(0, G)],
                    ldl_hbm.at[pl.ds(pl.multiple_of(wid * LCAP + off, 8), G)])
    cntv[...] = jnp.zeros((16,), jnp.int32) + (off + G)
    pltpu.sync_copy(cntv, cnt_hbm.at[pl.ds(pl.multiple_of(wid * 16, 8), 16)])


def _make_agg(D):
    @functools.partial(
        pl.kernel,
        out_type=jax.ShapeDtypeStruct((NPAD, D), jnp.float32),
        mesh=_MESH,
        compiler_params=pltpu.CompilerParams(
            needs_layout_passes=False, use_tc_tiling_on_sc=False),
        scratch_types=[
            pltpu.VMEM((NPT + 1, D), jnp.float32),
            pltpu.VMEM((G,), jnp.int32),
            pltpu.VMEM((G,), jnp.int32),
            pltpu.VMEM((G,), jnp.int32),
            pltpu.VMEM((G,), jnp.int32),
            pltpu.VMEM((G, D), jnp.float32),
            pltpu.VMEM((G, D), jnp.float32),
            pltpu.VMEM((16,), jnp.int32),
            pltpu.SemaphoreType.DMA,
            pltpu.SemaphoreType.DMA,
        ],
    )
    def _agg(vals_hbm, srcl_hbm, ldl_hbm, cnt_hbm, out_hbm,
             acc, idx0, idx1, ld0, ld1, row0, row1, cntv, sem0, sem1):
        wid = _wid()
        lo = wid * NPT
        inf16 = jnp.full((16,), jnp.inf, jnp.float32)

        def init_body(r, _):
            for c in range(D // 16):
                acc[r, pl.ds(c * 16, 16)] = inf16
            return 0

        lax.fori_loop(0, NPT + 1, init_body, 0)

        pltpu.sync_copy(cnt_hbm.at[pl.ds(pl.multiple_of(wid * 16, 8), 16)], cntv)
        nblk = cntv[pl.ds(0, 16)][0] // G

        slots = ((idx0, ld0, row0, sem0), (idx1, ld1, row1, sem1))

        def fire(b, slot):
            idxr, ldr, rowr, sem = slots[slot]
            pltpu.sync_copy(srcl_hbm.at[pl.ds(pl.multiple_of(wid * LCAP + b * G, 8), G)], idxr)
            pltpu.sync_copy(ldl_hbm.at[pl.ds(pl.multiple_of(wid * LCAP + b * G, 8), G)], ldr)
            pltpu.async_copy(vals_hbm.at[idxr], rowr, sem)

        def process(slot):
            idxr, ldr, rowr, sem = slots[slot]
            pltpu.make_async_copy(vals_hbm.at[idxr], rowr, sem).wait()

            def grp_body(jj, _):
                ldv = ldr[pl.ds(jj * 16, 16)]
                for k in range(16):
                    ld = ldv[k]
                    j = jj * 16 + k
                    for c in range(D // 16):
                        g = rowr[j, pl.ds(c * 16, 16)]
                        a = acc[ld, pl.ds(c * 16, 16)]
                        acc[ld, pl.ds(c * 16, 16)] = jnp.minimum(a, g)
                return 0

            lax.fori_loop(0, G // 16, grp_body, 0)

        fire(0, 0)

        def pair_body(bb, _):
            b0 = bb * 2

            @pl.when(b0 + 1 < nblk)
            def _():
                fire(b0 + 1, 1)

            process(0)

            @pl.when(b0 + 2 < nblk)
            def _():
                fire(b0 + 2, 0)

            @pl.when(b0 + 1 < nblk)
            def _():
                process(1)

            return 0

        lax.fori_loop(0, (nblk + 1) // 2, pair_body, 0)
        pltpu.sync_copy(acc.at[pl.ds(0, NPT)], out_hbm.at[pl.ds(lo, NPT)])

    return _agg


_agg128 = _make_agg(128)
_agg64 = _make_agg(64)
_agg16 = _make_agg(16)


def _row_mask():
    rows = lax.broadcasted_iota(jnp.int32, (NPAD, 1), 0)
    return (rows < N).astype(jnp.float32)


def _bn_relu(y, g, b, relu):
    mask = _row_mask()
    m = jnp.sum(y * mask, axis=0, keepdims=True) / N
    d = (y - m) * mask
    v = jnp.sum(d * d, axis=0, keepdims=True) / N
    z = (y - m) / jnp.sqrt(v + 1e-5) * g + b
    if relu:
        z = jnp.maximum(z, 0.0)
    return z


def _dense_stage(agg, h, Wl, bl, Wr, g, be, *, bn=True, relu=True):
    H = Wl.shape[1]

    def body(agg_ref, h_ref, Wl_ref, bl_ref, Wr_ref, g_ref, be_ref, o_ref):
        a = agg_ref[...]
        a = jnp.where(jnp.isfinite(a), a, 0.0)
        y = (jnp.dot(a, Wl_ref[...], preferred_element_type=jnp.float32)
             + jnp.dot(h_ref[...], Wr_ref[...], preferred_element_type=jnp.float32)
             + bl_ref[...])
        if bn:
            y = _bn_relu(y, g_ref[...], be_ref[...], relu)
        o_ref[...] = y

    return pl.pallas_call(
        body, out_shape=jax.ShapeDtypeStruct((NPAD, H), jnp.float32),
    )(agg, h, Wl, bl, Wr, g, be)


def _shared_stage(aggx, aggrm, x, rm16, Wls_a, Wls_b, Wrs_a, Wrs_b, bls, gs, bes):
    def body(aggx_ref, aggrm_ref, x_ref, rm_ref, Wla_ref, Wlb_ref, Wra_ref,
             Wrb_ref, bls_ref, gs_ref, bes_ref, o_ref):
        ax = aggx_ref[...]
        ax = jnp.where(jnp.isfinite(ax), ax, 0.0)
        ar = aggrm_ref[...]
        ar = jnp.where(jnp.isfinite(ar), ar, 0.0)
        y = (jnp.dot(ax, Wla_ref[...], preferred_element_type=jnp.float32)
             + jnp.dot(ar, Wlb_ref[...], preferred_element_type=jnp.float32)
             + jnp.dot(x_ref[...], Wra_ref[...], preferred_element_type=jnp.float32)
             + jnp.dot(rm_ref[...], Wrb_ref[...], preferred_element_type=jnp.float32)
             + bls_ref[...])
        o_ref[...] = _bn_relu(y, gs_ref[...], bes_ref[...], True)

    return pl.pallas_call(
        body, out_shape=jax.ShapeDtypeStruct((NPAD, 128), jnp.float32),
    )(aggx, aggrm, x, rm16, Wls_a, Wls_b, Wrs_a, Wrs_b, bls, gs, bes)


def _final_stage(aggxs, xs, rm16, Wlr, blr, Wrr, gr, ber, Wr3p, br3,
                 Wlm, blm, Wrm, gm, bem, Wm3p, bm3):
    def body(aggxs_ref, xs_ref, rm_ref, Wlr_ref, blr_ref, Wrr_ref, gr_ref,
             ber_ref, Wr3_ref, br3_ref, Wlm_ref, blm_ref, Wrm_ref, gm_ref,
             bem_ref, Wm3_ref, bm3_ref, lsm_ref, o_ref):
        a = aggxs_ref[...]
        a = jnp.where(jnp.isfinite(a), a, 0.0)
        xs = xs_ref[...]
        rm = rm_ref[...]

        cols = lax.broadcasted_iota(jnp.int32, (NPAD, 16), 1)
        cmask = cols < 3
        mx = jnp.max(jnp.where(cmask, rm, -jnp.inf), axis=1, keepdims=True)
        se = jnp.sum(jnp.where(cmask, jnp.exp(rm - mx), 0.0), axis=1,
                     keepdims=True)
        lsm_ref[...] = rm - (mx + jnp.log(se))
        a0 = rm[:, 0:1]
        a1 = rm[:, 1:2]
        a2 = rm[:, 2:3]
        labels = jnp.where((a2 > a0) & (a2 > a1), 1.0, 0.0)

        yr = (jnp.dot(a, Wlr_ref[...], preferred_element_type=jnp.float32)
              + jnp.dot(xs, Wrr_ref[...], preferred_element_type=jnp.float32)
              + blr_ref[...])
        hr = _bn_relu(yr, gr_ref[...], ber_ref[...], True)
        rt = (jnp.dot(hr, Wr3_ref[...], preferred_element_type=jnp.float32)
              + br3_ref[...])

        ym = (jnp.dot(a, Wlm_ref[...], preferred_element_type=jnp.float32)
              + jnp.dot(xs, Wrm_ref[...], preferred_element_type=jnp.float32)
              + blm_ref[...])
        hm = _bn_relu(ym, gm_ref[...], bem_ref[...], True)
        md = (jnp.dot(hm, Wm3_ref[...], preferred_element_type=jnp.float32)
              + bm3_ref[...])

        ocols = lax.broadcasted_iota(jnp.int32, (NPAD, 128), 1)
        o_ref[...] = (jnp.where(ocols == 0, rt, 0.0)
                      + jnp.where(ocols == 1, md, 0.0)) * labels

    return pl.pallas_call(
        body,
        out_shape=(
            jax.ShapeDtypeStruct((NPAD, 16), jnp.float32),
            jax.ShapeDtypeStruct((NPAD, 128), jnp.float32),
        ),
    )(aggxs, xs, rm16, Wlr, blr, Wrr, gr, ber, Wr3p, br3, Wlm, blm, Wrm, gm,
      bem, Wm3p, bm3)


def kernel(x, edge_index, Wl1, bl1, Wr1, g1, be1, Wl2, bl2, Wr2, g2, be2,
           Wl4, bl4, Wr4, Wls, bls, Wrs, gs, bes, Wlr, blr, Wrr, gr, ber,
           Wr3, br3, Wlm, blm, Wrm, gm, bem, Wm3, bm3):
    src = edge_index[0]
    dst = edge_index[1]

    x_pad = jnp.zeros((NPAD, 128), jnp.float32).at[:N].set(x)

    Wls_a, Wls_b3 = Wls[:128], Wls[128:]
    Wrs_a, Wrs_b3 = Wrs[:128], Wrs[128:]
    Wls_b = jnp.zeros((16, 128), jnp.float32).at[:3].set(Wls_b3)
    Wrs_b = jnp.zeros((16, 128), jnp.float32).at[:3].set(Wrs_b3)
    Wl4p = jnp.pad(Wl4, ((0, 0), (0, 13)))
    Wr4p = jnp.pad(Wr4, ((0, 0), (0, 13)))
    bl4p = jnp.pad(bl4, (0, 13))
    Wr3p = jnp.pad(Wr3, ((0, 0), (0, 127)))
    Wm3p = jnp.pad(Wm3, ((0, 0), (1, 126)))

    srcl, ldl, cnts = _bin_edges(src, dst)

    aggx = _agg128(x_pad, srcl, ldl, cnts)
    h1 = _dense_stage(aggx, x_pad, Wl1, bl1, Wr1, g1, be1)
    aggh1 = _agg128(h1, srcl, ldl, cnts)
    h2 = _dense_stage(aggh1, h1, Wl2, bl2, Wr2, g2, be2)
    aggh2 = _agg64(h2, srcl, ldl, cnts)
    rm16 = _dense_nobn(aggh2, h2, Wl4p, bl4p, Wr4p)
    aggrm = _agg16(rm16, srcl, ldl, cnts)
    xs = _shared_stage(aggx, aggrm, x_pad, rm16, Wls_a, Wls_b, Wrs_a, Wrs_b,
                       bls, gs, bes)
    aggxs = _agg128(xs, srcl, ldl, cnts)
    lsm, out2 = _final_stage(aggxs, xs, rm16, Wlr, blr, Wrr, gr, ber, Wr3p,
                             br3, Wlm, blm, Wrm, gm, bem, Wm3p, bm3)

    rm_out = lsm[:N, :3]
    rt = out2[:N, 0]
    md = out2[:N, 1]
    return (rm_out, rt, md)


def _dense_nobn(agg, h, Wl, bl, Wr):
    H = Wl.shape[1]

    def body(agg_ref, h_ref, Wl_ref, bl_ref, Wr_ref, o_ref):
        a = agg_ref[...]
        a = jnp.where(jnp.isfinite(a), a, 0.0)
        o_ref[...] = (jnp.dot(a, Wl_ref[...], preferred_element_type=jnp.float32)
                      + jnp.dot(h_ref[...], Wr_ref[...],
                                preferred_element_type=jnp.float32)
                      + bl_ref[...])

    return pl.pallas_call(
        body, out_shape=jax.ShapeDtypeStruct((NPAD, H), jnp.float32),
    )(agg, h, Wl, bl, Wr)


# speedup vs baseline: 4.1575x; 1.3812x over previous
import functools

import jax
import jax.numpy as jnp
from jax import lax
from jax.experimental import pallas as pl
from jax.experimental.pallas import tpu as pltpu
from jax.experimental.pallas import tpu_sc as plsc

N = 10000
NT = 32
NPT = 320
NPAD = NT * NPT
E = 320000
CHUNK = 2560
G = 128
LCAP = E + 1536

_MESH = plsc.VectorSubcoreMesh(core_axis_name="c", subcore_axis_name="s")


def _wid():
    return lax.axis_index("s") * 2 + lax.axis_index("c")


NCH = E // CHUNK
SS = 528
FT = 384


@functools.partial(
    pl.kernel,
    out_type=(
        jax.ShapeDtypeStruct((NT * LCAP,), jnp.int32),
        jax.ShapeDtypeStruct((NT * LCAP,), jnp.int32),
        jax.ShapeDtypeStruct((NT * 16,), jnp.int32),
    ),
    mesh=_MESH,
    compiler_params=pltpu.CompilerParams(needs_layout_passes=False),
    scratch_types=[
        pltpu.VMEM((CHUNK,), jnp.int32),
        pltpu.VMEM((CHUNK,), jnp.int32),
        pltpu.VMEM((CHUNK,), jnp.int32),
        pltpu.VMEM((CHUNK,), jnp.int32),
        pltpu.VMEM((SS,), jnp.int32),
        pltpu.VMEM((SS,), jnp.int32),
        pltpu.VMEM((16,), jnp.int32),
        pltpu.SemaphoreType.DMA,
        pltpu.SemaphoreType.DMA,
    ],
)
def _bin_edges(src_hbm, dst_hbm, srcl_hbm, ldl_hbm, cnt_hbm,
               srcb0, srcb1, dstb0, dstb1, sstage, lstage, cntv, sem0, sem1):
    wid = _wid()
    lo = wid * NPT
    hi = lo + NPT
    iota16 = lax.iota(jnp.int32, 16)
    slots = ((srcb0, dstb0, sem0), (srcb1, dstb1, sem1))

    def fire(k, slot):
        sb, db, sem = slots[slot]
        pltpu.async_copy(src_hbm.at[pl.ds(k * CHUNK, CHUNK)], sb, sem)
        pltpu.async_copy(dst_hbm.at[pl.ds(k * CHUNK, CHUNK)], db, sem)

    def _flush(c):
        posv, off = c
        base = pl.multiple_of(wid * LCAP + off, 8)
        pltpu.sync_copy(sstage.at[pl.ds(0, FT)], srcl_hbm.at[pl.ds(base, FT)])
        pltpu.sync_copy(lstage.at[pl.ds(0, FT)], ldl_hbm.at[pl.ds(base, FT)])
        for j in range(8):
            sstage[pl.ds(j * 16, 16)] = sstage[pl.ds(FT + j * 16, 16)]
            lstage[pl.ds(j * 16, 16)] = lstage[pl.ds(FT + j * 16, 16)]
        return (posv - FT, off + FT)

    def process(slot, carry):
        sb, db, sem = slots[slot]
        pltpu.make_async_copy(src_hbm.at[pl.ds(0, CHUNK)], sb, sem).wait()
        pltpu.make_async_copy(dst_hbm.at[pl.ds(0, CHUNK)], db, sem).wait()

        def grp(g, c):
            posv, off = c
            for u in range(8):
                o = g * 128 + u * 16
                dv = db[pl.ds(o, 16)]
                sv = sb[pl.ds(o, 16)]
                m = (dv >= lo) & (dv < hi)
                m32 = m.astype(jnp.int32)
                cs = jnp.cumsum(m32)
                pcv = plsc.all_reduce_population_count(m)
                positions = posv + (cs - m32)
                plsc.store_scatter(sstage, [positions], sv, mask=m)
                plsc.store_scatter(lstage, [positions], dv - lo, mask=m)
                posv = posv + pcv
            pos0 = posv[0]
            return lax.cond(pos0 >= FT, _flush, lambda c_: c_, (posv, off))

        return lax.fori_loop(0, CHUNK // 128, grp, carry)

    fire(0, 0)

    def pair_body(kk, carry):
        k0 = kk * 2

        @pl.when(k0 + 1 < NCH)
        def _():
            fire(k0 + 1, 1)

        carry = process(0, carry)

        @pl.when(k0 + 2 < NCH)
        def _():
            fire(k0 + 2, 0)

        carry = lax.cond(k0 + 1 < NCH,
                         lambda c: process(1, c), lambda c: c, carry)
        return carry

    posv, off = lax.fori_loop(0, (NCH + 1) // 2, pair_body,
                              (jnp.zeros((16,), jnp.int32), 0))

    pos0 = posv[0]
    padsrc = lo + iota16
    padld = jnp.zeros((16,), jnp.int32) + NPT
    for j16 in range(SS // 16):
        idxs = j16 * 16 + iota16
        mpad = idxs >= pos0
        plsc.store_scatter(sstage, [idxs], padsrc, mask=mpad)
        plsc.store_scatter(lstage, [idxs], padld, mask=mpad)
    nb = jnp.maximum((pos0 + G - 1) // G, 1)

    def wb(i, o):
        base = pl.multiple_of(wid * LCAP + off + i * G, 8)
        pltpu.sync_copy(sstage.at[pl.ds(i * G, G)], srcl_hbm.at[pl.ds(base, G)])
        pltpu.sync_copy(lstage.at[pl.ds(i * G, G)], ldl_hbm.at[pl.ds(base, G)])
        return o

    lax.fori_loop(0, nb, wb, 0)
    cntv[...] = jnp.zeros((16,), jnp.int32) + (off + nb * G)
    pltpu.sync_copy(cntv, cnt_hbm.at[pl.ds(pl.multiple_of(wid * 16, 8), 16)])


SB = 1024


def _make_agg(D):
    @functools.partial(
        pl.kernel,
        out_type=jax.ShapeDtypeStruct((NPAD, D), jnp.float32),
        mesh=_MESH,
        compiler_params=pltpu.CompilerParams(
            needs_layout_passes=False, use_tc_tiling_on_sc=False),
        scratch_types=[
            pltpu.VMEM((NPT + 1, D), jnp.float32),
            pltpu.VMEM((SB,), jnp.int32),
            pltpu.VMEM((SB,), jnp.int32),
            pltpu.VMEM((SB,), jnp.int32),
            pltpu.VMEM((SB,), jnp.int32),
            pltpu.VMEM((G, D), jnp.float32),
            pltpu.VMEM((G, D), jnp.float32),
            pltpu.VMEM((16,), jnp.int32),
            pltpu.SemaphoreType.DMA,
            pltpu.SemaphoreType.DMA,
            pltpu.SemaphoreType.DMA,
            pltpu.SemaphoreType.DMA,
        ],
    )
    def _agg(vals_hbm, srcl_hbm, ldl_hbm, cnt_hbm, out_hbm,
             acc, sidx0, sidx1, sld0, sld1, row0, row1, cntv,
             lsem0, lsem1, gsem0, gsem1):
        wid = _wid()
        lo = wid * NPT
        inf16 = jnp.full((16,), jnp.inf, jnp.float32)

        def init_body(r, _):
            for c in range(D // 16):
                acc[r, pl.ds(c * 16, 16)] = inf16
            return 0

        lax.fori_loop(0, NPT + 1, init_body, 0)

        pltpu.sync_copy(cnt_hbm.at[pl.ds(pl.multiple_of(wid * 16, 8), 16)],
                        cntv)
        nblk = cntv[pl.ds(0, 16)][0] // G
        nsb = (nblk + 7) // 8

        lslots = ((sidx0, sld0, lsem0), (sidx1, sld1, lsem1))
        rslots = ((row0, gsem0), (row1, gsem1))

        def fire_lists(sb, slot):
            si, sl, sem = lslots[slot]
            base = pl.multiple_of(wid * LCAP + sb * SB, 8)
            pltpu.async_copy(srcl_hbm.at[pl.ds(base, SB)], si, sem)
            pltpu.async_copy(ldl_hbm.at[pl.ds(base, SB)], sl, sem)

        def wait_lists(slot):
            si, sl, sem = lslots[slot]
            pltpu.make_async_copy(srcl_hbm.at[pl.ds(0, SB)], si, sem).wait()
            pltpu.make_async_copy(ldl_hbm.at[pl.ds(0, SB)], sl, sem).wait()

        def fire_g(lslot, bl, rslot):
            si, _, _ = lslots[lslot]
            rbuf, gsem = rslots[rslot]
            pltpu.async_copy(vals_hbm.at[si.at[pl.ds(bl * G, G)]], rbuf, gsem)

        def process_rows(lslot, bl, rslot):
            si, sl, _ = lslots[lslot]
            rbuf, gsem = rslots[rslot]
            pltpu.make_async_copy(vals_hbm.at[si.at[pl.ds(0, G)]], rbuf,
                                  gsem).wait()

            def grp_body(jj, _):
                ldv = sl[pl.ds(bl * G + jj * 16, 16)]
                for k in range(16):
                    ld = ldv[k]
                    j = jj * 16 + k
                    for c in range(D // 16):
                        g = rbuf[j, pl.ds(c * 16, 16)]
                        a = acc[ld, pl.ds(c * 16, 16)]
                        acc[ld, pl.ds(c * 16, 16)] = jnp.minimum(a, g)
                return 0

            lax.fori_loop(0, G // 16, grp_body, 0)

        def do_superblock(sb, lslot):
            nloc = jnp.minimum(8, nblk - sb * 8)
            fire_g(lslot, 0, 0)

            def pair(bb, _):
                b0 = bb * 2

                @pl.when(b0 + 1 < nloc)
                def _():
                    fire_g(lslot, b0 + 1, 1)

                process_rows(lslot, b0, 0)

                @pl.when(b0 + 2 < nloc)
                def _():
                    fire_g(lslot, b0 + 2, 0)

                @pl.when(b0 + 1 < nloc)
                def _():
                    process_rows(lslot, b0 + 1, 1)

                return 0

            lax.fori_loop(0, (nloc + 1) // 2, pair, 0)

        fire_lists(0, 0)

        def sb_pair(ss, _):
            s0 = ss * 2

            @pl.when(s0 + 1 < nsb)
            def _():
                fire_lists(s0 + 1, 1)

            wait_lists(0)
            do_superblock(s0, 0)

            @pl.when(s0 + 2 < nsb)
            def _():
                fire_lists(s0 + 2, 0)

            @pl.when(s0 + 1 < nsb)
            def _():
                wait_lists(1)
                do_superblock(s0 + 1, 1)

            return 0

        lax.fori_loop(0, (nsb + 1) // 2, sb_pair, 0)
        pltpu.sync_copy(acc.at[pl.ds(0, NPT)], out_hbm.at[pl.ds(lo, NPT)])

    return _agg


_agg128 = _make_agg(128)
_agg64 = _make_agg(64)
_agg16 = _make_agg(16)


def _row_mask():
    rows = lax.broadcasted_iota(jnp.int32, (NPAD, 1), 0)
    return (rows < N).astype(jnp.float32)


def _bn_relu(y, g, b, relu):
    mask = _row_mask()
    m = jnp.sum(y * mask, axis=0, keepdims=True) / N
    d = (y - m) * mask
    v = jnp.sum(d * d, axis=0, keepdims=True) / N
    z = (y - m) / jnp.sqrt(v + 1e-5) * g + b
    if relu:
        z = jnp.maximum(z, 0.0)
    return z


def _dense_stage(agg, h, Wl, bl, Wr, g, be, *, bn=True, relu=True):
    H = Wl.shape[1]

    def body(agg_ref, h_ref, Wl_ref, bl_ref, Wr_ref, g_ref, be_ref, o_ref):
        a = agg_ref[...]
        a = jnp.where(jnp.isfinite(a), a, 0.0)
        y = (jnp.dot(a, Wl_ref[...], preferred_element_type=jnp.float32)
             + jnp.dot(h_ref[...], Wr_ref[...], preferred_element_type=jnp.float32)
             + bl_ref[...])
        if bn:
            y = _bn_relu(y, g_ref[...], be_ref[...], relu)
        o_ref[...] = y

    return pl.pallas_call(
        body, out_shape=jax.ShapeDtypeStruct((NPAD, H), jnp.float32),
    )(agg, h, Wl, bl, Wr, g, be)


def _shared_stage(aggx, aggrm, x, rm16, Wls_a, Wls_b, Wrs_a, Wrs_b, bls, gs, bes):
    def body(aggx_ref, aggrm_ref, x_ref, rm_ref, Wla_ref, Wlb_ref, Wra_ref,
             Wrb_ref, bls_ref, gs_ref, bes_ref, o_ref):
        ax = aggx_ref[...]
        ax = jnp.where(jnp.isfinite(ax), ax, 0.0)
        ar = aggrm_ref[...]
        ar = jnp.where(jnp.isfinite(ar), ar, 0.0)
        y = (jnp.dot(ax, Wla_ref[...], preferred_element_type=jnp.float32)
             + jnp.dot(ar, Wlb_ref[...], preferred_element_type=jnp.float32)
             + jnp.dot(x_ref[...], Wra_ref[...], preferred_element_type=jnp.float32)
             + jnp.dot(rm_ref[...], Wrb_ref[...], preferred_element_type=jnp.float32)
             + bls_ref[...])
        o_ref[...] = _bn_relu(y, gs_ref[...], bes_ref[...], True)

    return pl.pallas_call(
        body, out_shape=jax.ShapeDtypeStruct((NPAD, 128), jnp.float32),
    )(aggx, aggrm, x, rm16, Wls_a, Wls_b, Wrs_a, Wrs_b, bls, gs, bes)


def _final_stage(aggxs, xs, rm16, Wlr, blr, Wrr, gr, ber, Wr3p, br3,
                 Wlm, blm, Wrm, gm, bem, Wm3p, bm3):
    def body(aggxs_ref, xs_ref, rm_ref, Wlr_ref, blr_ref, Wrr_ref, gr_ref,
             ber_ref, Wr3_ref, br3_ref, Wlm_ref, blm_ref, Wrm_ref, gm_ref,
             bem_ref, Wm3_ref, bm3_ref, lsm_ref, o_ref):
        a = aggxs_ref[...]
        a = jnp.where(jnp.isfinite(a), a, 0.0)
        xs = xs_ref[...]
        rm = rm_ref[...]

        cols = lax.broadcasted_iota(jnp.int32, (NPAD, 16), 1)
        cmask = cols < 3
        mx = jnp.max(jnp.where(cmask, rm, -jnp.inf), axis=1, keepdims=True)
        se = jnp.sum(jnp.where(cmask, jnp.exp(rm - mx), 0.0), axis=1,
                     keepdims=True)
        lsm_ref[...] = rm - (mx + jnp.log(se))
        a0 = rm[:, 0:1]
        a1 = rm[:, 1:2]
        a2 = rm[:, 2:3]
        labels = jnp.where((a2 > a0) & (a2 > a1), 1.0, 0.0)

        yr = (jnp.dot(a, Wlr_ref[...], preferred_element_type=jnp.float32)
              + jnp.dot(xs, Wrr_ref[...], preferred_element_type=jnp.float32)
              + blr_ref[...])
        hr = _bn_relu(yr, gr_ref[...], ber_ref[...], True)
        rt = (jnp.dot(hr, Wr3_ref[...], preferred_element_type=jnp.float32)
              + br3_ref[...])

        ym = (jnp.dot(a, Wlm_ref[...], preferred_element_type=jnp.float32)
              + jnp.dot(xs, Wrm_ref[...], preferred_element_type=jnp.float32)
              + blm_ref[...])
        hm = _bn_relu(ym, gm_ref[...], bem_ref[...], True)
        md = (jnp.dot(hm, Wm3_ref[...], preferred_element_type=jnp.float32)
              + bm3_ref[...])

        ocols = lax.broadcasted_iota(jnp.int32, (NPAD, 128), 1)
        o_ref[...] = (jnp.where(ocols == 0, rt, 0.0)
                      + jnp.where(ocols == 1, md, 0.0)) * labels

    return pl.pallas_call(
        body,
        out_shape=(
            jax.ShapeDtypeStruct((NPAD, 16), jnp.float32),
            jax.ShapeDtypeStruct((NPAD, 128), jnp.float32),
        ),
    )(aggxs, xs, rm16, Wlr, blr, Wrr, gr, ber, Wr3p, br3, Wlm, blm, Wrm, gm,
      bem, Wm3p, bm3)


def kernel(x, edge_index, Wl1, bl1, Wr1, g1, be1, Wl2, bl2, Wr2, g2, be2,
           Wl4, bl4, Wr4, Wls, bls, Wrs, gs, bes, Wlr, blr, Wrr, gr, ber,
           Wr3, br3, Wlm, blm, Wrm, gm, bem, Wm3, bm3):
    src = edge_index[0]
    dst = edge_index[1]

    x_pad = jnp.zeros((NPAD, 128), jnp.float32).at[:N].set(x)

    Wls_a, Wls_b3 = Wls[:128], Wls[128:]
    Wrs_a, Wrs_b3 = Wrs[:128], Wrs[128:]
    Wls_b = jnp.zeros((16, 128), jnp.float32).at[:3].set(Wls_b3)
    Wrs_b = jnp.zeros((16, 128), jnp.float32).at[:3].set(Wrs_b3)
    Wl4p = jnp.pad(Wl4, ((0, 0), (0, 13)))
    Wr4p = jnp.pad(Wr4, ((0, 0), (0, 13)))
    bl4p = jnp.pad(bl4, (0, 13))
    Wr3p = jnp.pad(Wr3, ((0, 0), (0, 127)))
    Wm3p = jnp.pad(Wm3, ((0, 0), (1, 126)))

    srcl, ldl, cnts = _bin_edges(src, dst)

    aggx = _agg128(x_pad, srcl, ldl, cnts)
    h1 = _dense_stage(aggx, x_pad, Wl1, bl1, Wr1, g1, be1)
    aggh1 = _agg128(h1, srcl, ldl, cnts)
    h2 = _dense_stage(aggh1, h1, Wl2, bl2, Wr2, g2, be2)
    aggh2 = _agg64(h2, srcl, ldl, cnts)
    rm16 = _dense_nobn(aggh2, h2, Wl4p, bl4p, Wr4p)
    aggrm = _agg16(rm16, srcl, ldl, cnts)
    xs = _shared_stage(aggx, aggrm, x_pad, rm16, Wls_a, Wls_b, Wrs_a, Wrs_b,
                       bls, gs, bes)
    aggxs = _agg128(xs, srcl, ldl, cnts)
    lsm, out2 = _final_stage(aggxs, xs, rm16, Wlr, blr, Wrr, gr, ber, Wr3p,
                             br3, Wlm, blm, Wrm, gm, bem, Wm3p, bm3)

    rm_out = lsm[:N, :3]
    rt = out2[:N, 0]
    md = out2[:N, 1]
    return (rm_out, rt, md)


def _dense_nobn(agg, h, Wl, bl, Wr):
    H = Wl.shape[1]

    def body(agg_ref, h_ref, Wl_ref, bl_ref, Wr_ref, o_ref):
        a = agg_ref[...]
        a = jnp.where(jnp.isfinite(a), a, 0.0)
        o_ref[...] = (jnp.dot(a, Wl_ref[...], preferred_element_type=jnp.float32)
                      + jnp.dot(h_ref[...], Wr_ref[...],
                                preferred_element_type=jnp.float32)
                      + bl_ref[...])

    return pl.pallas_call(
        body, out_shape=jax.ShapeDtypeStruct((NPAD, H), jnp.float32),
    )(agg, h, Wl, bl, Wr)


# speedup vs baseline: 5.3636x; 1.2901x over previous
import functools

import jax
import jax.numpy as jnp
from jax import lax
from jax.experimental import pallas as pl
from jax.experimental.pallas import tpu as pltpu
from jax.experimental.pallas import tpu_sc as plsc

N = 10000
NT = 32
NPT = 320
NPAD = NT * NPT
E = 320000
CHUNK = 2560
G = 128
LCAP = E + 1536

_MESH = plsc.VectorSubcoreMesh(core_axis_name="c", subcore_axis_name="s")


def _wid():
    return lax.axis_index("s") * 2 + lax.axis_index("c")


NCH = E // CHUNK
SS = 528
FT = 384


@functools.partial(
    pl.kernel,
    out_type=(
        jax.ShapeDtypeStruct((NT * LCAP,), jnp.int32),
        jax.ShapeDtypeStruct((NT * LCAP,), jnp.int32),
        jax.ShapeDtypeStruct((NT * 16,), jnp.int32),
    ),
    mesh=_MESH,
    compiler_params=pltpu.CompilerParams(needs_layout_passes=False),
    scratch_types=[
        pltpu.VMEM((CHUNK,), jnp.int32),
        pltpu.VMEM((CHUNK,), jnp.int32),
        pltpu.VMEM((CHUNK,), jnp.int32),
        pltpu.VMEM((CHUNK,), jnp.int32),
        pltpu.VMEM((SS,), jnp.int32),
        pltpu.VMEM((SS,), jnp.int32),
        pltpu.VMEM((16,), jnp.int32),
        pltpu.SemaphoreType.DMA,
        pltpu.SemaphoreType.DMA,
    ],
)
def _bin_edges(src_hbm, dst_hbm, srcl_hbm, ldl_hbm, cnt_hbm,
               srcb0, srcb1, dstb0, dstb1, sstage, lstage, cntv, sem0, sem1):
    wid = _wid()
    lo = wid * NPT
    hi = lo + NPT
    iota16 = lax.iota(jnp.int32, 16)
    slots = ((srcb0, dstb0, sem0), (srcb1, dstb1, sem1))

    def fire(k, slot):
        sb, db, sem = slots[slot]
        pltpu.async_copy(src_hbm.at[pl.ds(k * CHUNK, CHUNK)], sb, sem)
        pltpu.async_copy(dst_hbm.at[pl.ds(k * CHUNK, CHUNK)], db, sem)

    def _flush(c):
        posv, off = c
        base = pl.multiple_of(wid * LCAP + off, 8)
        pltpu.sync_copy(sstage.at[pl.ds(0, FT)], srcl_hbm.at[pl.ds(base, FT)])
        pltpu.sync_copy(lstage.at[pl.ds(0, FT)], ldl_hbm.at[pl.ds(base, FT)])
        for j in range(8):
            sstage[pl.ds(j * 16, 16)] = sstage[pl.ds(FT + j * 16, 16)]
            lstage[pl.ds(j * 16, 16)] = lstage[pl.ds(FT + j * 16, 16)]
        return (posv - FT, off + FT)

    def process(slot, carry):
        sb, db, sem = slots[slot]
        pltpu.make_async_copy(src_hbm.at[pl.ds(0, CHUNK)], sb, sem).wait()
        pltpu.make_async_copy(dst_hbm.at[pl.ds(0, CHUNK)], db, sem).wait()

        def grp(g, c):
            posv, off = c
            for u in range(8):
                o = g * 128 + u * 16
                dv = db[pl.ds(o, 16)]
                sv = sb[pl.ds(o, 16)]
                m = (dv >= lo) & (dv < hi)
                m32 = m.astype(jnp.int32)
                cs = jnp.cumsum(m32)
                pcv = plsc.all_reduce_population_count(m)
                positions = posv + (cs - m32)
                plsc.store_scatter(sstage, [positions], sv, mask=m)
                plsc.store_scatter(lstage, [positions], dv - lo, mask=m)
                posv = posv + pcv
            pos0 = posv[0]
            return lax.cond(pos0 >= FT, _flush, lambda c_: c_, (posv, off))

        return lax.fori_loop(0, CHUNK // 128, grp, carry)

    fire(0, 0)

    def pair_body(kk, carry):
        k0 = kk * 2

        @pl.when(k0 + 1 < NCH)
        def _():
            fire(k0 + 1, 1)

        carry = process(0, carry)

        @pl.when(k0 + 2 < NCH)
        def _():
            fire(k0 + 2, 0)

        carry = lax.cond(k0 + 1 < NCH,
                         lambda c: process(1, c), lambda c: c, carry)
        return carry

    posv, off = lax.fori_loop(0, (NCH + 1) // 2, pair_body,
                              (jnp.zeros((16,), jnp.int32), 0))

    pos0 = posv[0]
    padsrc = lo + iota16
    padld = jnp.zeros((16,), jnp.int32) + NPT
    for j16 in range(SS // 16):
        idxs = j16 * 16 + iota16
        mpad = idxs >= pos0
        plsc.store_scatter(sstage, [idxs], padsrc, mask=mpad)
        plsc.store_scatter(lstage, [idxs], padld, mask=mpad)
    nb = jnp.maximum((pos0 + G - 1) // G, 1)

    def wb(i, o):
        base = pl.multiple_of(wid * LCAP + off + i * G, 8)
        pltpu.sync_copy(sstage.at[pl.ds(i * G, G)], srcl_hbm.at[pl.ds(base, G)])
        pltpu.sync_copy(lstage.at[pl.ds(i * G, G)], ldl_hbm.at[pl.ds(base, G)])
        return o

    lax.fori_loop(0, nb, wb, 0)
    cntv[...] = jnp.zeros((16,), jnp.int32) + (off + nb * G)
    pltpu.sync_copy(cntv, cnt_hbm.at[pl.ds(pl.multiple_of(wid * 16, 8), 16)])


SB = 1024


def _make_agg(D):
    @functools.partial(
        pl.kernel,
        out_type=jax.ShapeDtypeStruct((NPAD, D), jnp.float32),
        mesh=_MESH,
        compiler_params=pltpu.CompilerParams(
            needs_layout_passes=False, use_tc_tiling_on_sc=False),
        scratch_types=[
            pltpu.VMEM((NPT + 1, D), jnp.float32),
            pltpu.VMEM((NPT + 1, D), jnp.float32),
            pltpu.VMEM((SB,), jnp.int32),
            pltpu.VMEM((SB,), jnp.int32),
            pltpu.VMEM((SB,), jnp.int32),
            pltpu.VMEM((SB,), jnp.int32),
            pltpu.VMEM((G, D), jnp.float32),
            pltpu.VMEM((G, D), jnp.float32),
            pltpu.VMEM((16,), jnp.int32),
            pltpu.SemaphoreType.DMA,
            pltpu.SemaphoreType.DMA,
            pltpu.SemaphoreType.DMA,
            pltpu.SemaphoreType.DMA,
        ],
    )
    def _agg(vals_hbm, srcl_hbm, ldl_hbm, cnt_hbm, out_hbm,
             acc0, acc1, sidx0, sidx1, sld0, sld1, row0, row1, cntv,
             lsem0, lsem1, gsem0, gsem1):
        wid = _wid()
        lo = wid * NPT
        inf16 = jnp.full((16,), jnp.inf, jnp.float32)

        def init_body(r, _):
            for c in range(D // 16):
                acc0[r, pl.ds(c * 16, 16)] = inf16
                acc1[r, pl.ds(c * 16, 16)] = inf16
            return 0

        lax.fori_loop(0, NPT + 1, init_body, 0)

        pltpu.sync_copy(cnt_hbm.at[pl.ds(pl.multiple_of(wid * 16, 8), 16)],
                        cntv)
        nblk = cntv[pl.ds(0, 16)][0] // G
        nsb = (nblk + 7) // 8

        lslots = ((sidx0, sld0, lsem0), (sidx1, sld1, lsem1))
        rslots = ((row0, gsem0), (row1, gsem1))

        def fire_lists(sb, slot):
            si, sl, sem = lslots[slot]
            base = pl.multiple_of(wid * LCAP + sb * SB, 8)
            pltpu.async_copy(srcl_hbm.at[pl.ds(base, SB)], si, sem)
            pltpu.async_copy(ldl_hbm.at[pl.ds(base, SB)], sl, sem)

        def wait_lists(slot):
            si, sl, sem = lslots[slot]
            pltpu.make_async_copy(srcl_hbm.at[pl.ds(0, SB)], si, sem).wait()
            pltpu.make_async_copy(ldl_hbm.at[pl.ds(0, SB)], sl, sem).wait()

        def fire_g(lslot, bl, rslot):
            si, _, _ = lslots[lslot]
            rbuf, gsem = rslots[rslot]
            pltpu.async_copy(vals_hbm.at[si.at[pl.ds(bl * G, G)]], rbuf, gsem)

        def process_rows(lslot, bl, rslot):
            si, sl, _ = lslots[lslot]
            rbuf, gsem = rslots[rslot]
            pltpu.make_async_copy(vals_hbm.at[si.at[pl.ds(0, G)]], rbuf,
                                  gsem).wait()

            def grp_body(jj, _):
                ldv = sl[pl.ds(bl * G + jj * 16, 16)]
                for k in range(0, 16, 2):
                    ld0 = ldv[k]
                    ld1 = ldv[k + 1]
                    j0 = jj * 16 + k
                    j1 = j0 + 1
                    for c in range(D // 16):
                        cs_ = pl.ds(c * 16, 16)
                        g0 = rbuf[j0, cs_]
                        a0 = acc0[ld0, cs_]
                        g1 = rbuf[j1, cs_]
                        a1 = acc1[ld1, cs_]
                        acc0[ld0, cs_] = jnp.minimum(a0, g0)
                        acc1[ld1, cs_] = jnp.minimum(a1, g1)
                return 0

            lax.fori_loop(0, G // 16, grp_body, 0)

        def do_superblock(sb, lslot):
            nloc = jnp.minimum(8, nblk - sb * 8)
            fire_g(lslot, 0, 0)

            def pair(bb, _):
                b0 = bb * 2

                @pl.when(b0 + 1 < nloc)
                def _():
                    fire_g(lslot, b0 + 1, 1)

                process_rows(lslot, b0, 0)

                @pl.when(b0 + 2 < nloc)
                def _():
                    fire_g(lslot, b0 + 2, 0)

                @pl.when(b0 + 1 < nloc)
                def _():
                    process_rows(lslot, b0 + 1, 1)

                return 0

            lax.fori_loop(0, (nloc + 1) // 2, pair, 0)

        fire_lists(0, 0)

        def sb_pair(ss, _):
            s0 = ss * 2

            @pl.when(s0 + 1 < nsb)
            def _():
                fire_lists(s0 + 1, 1)

            wait_lists(0)
            do_superblock(s0, 0)

            @pl.when(s0 + 2 < nsb)
            def _():
                fire_lists(s0 + 2, 0)

            @pl.when(s0 + 1 < nsb)
            def _():
                wait_lists(1)
                do_superblock(s0 + 1, 1)

            return 0

        lax.fori_loop(0, (nsb + 1) // 2, sb_pair, 0)

        def merge_body(r, _):
            for c in range(D // 16):
                cs_ = pl.ds(c * 16, 16)
                acc0[r, cs_] = jnp.minimum(acc0[r, cs_], acc1[r, cs_])
            return 0

        lax.fori_loop(0, NPT, merge_body, 0)
        pltpu.sync_copy(acc0.at[pl.ds(0, NPT)], out_hbm.at[pl.ds(lo, NPT)])

    return _agg


_agg128 = _make_agg(128)
_agg64 = _make_agg(64)
_agg16 = _make_agg(16)


def _row_mask():
    rows = lax.broadcasted_iota(jnp.int32, (NPAD, 1), 0)
    return (rows < N).astype(jnp.float32)


def _bn_relu(y, g, b, relu):
    mask = _row_mask()
    m = jnp.sum(y * mask, axis=0, keepdims=True) / N
    d = (y - m) * mask
    v = jnp.sum(d * d, axis=0, keepdims=True) / N
    z = (y - m) / jnp.sqrt(v + 1e-5) * g + b
    if relu:
        z = jnp.maximum(z, 0.0)
    return z


def _dense_stage(agg, h, Wl, bl, Wr, g, be, *, bn=True, relu=True):
    H = Wl.shape[1]

    def body(agg_ref, h_ref, Wl_ref, bl_ref, Wr_ref, g_ref, be_ref, o_ref):
        a = agg_ref[...]
        a = jnp.where(jnp.isfinite(a), a, 0.0)
        y = (jnp.dot(a, Wl_ref[...], preferred_element_type=jnp.float32)
             + jnp.dot(h_ref[...], Wr_ref[...], preferred_element_type=jnp.float32)
             + bl_ref[...])
        if bn:
            y = _bn_relu(y, g_ref[...], be_ref[...], relu)
        o_ref[...] = y

    return pl.pallas_call(
        body, out_shape=jax.ShapeDtypeStruct((NPAD, H), jnp.float32),
    )(agg, h, Wl, bl, Wr, g, be)


def _shared_stage(aggx, aggrm, x, rm16, Wls_a, Wls_b, Wrs_a, Wrs_b, bls, gs, bes):
    def body(aggx_ref, aggrm_ref, x_ref, rm_ref, Wla_ref, Wlb_ref, Wra_ref,
             Wrb_ref, bls_ref, gs_ref, bes_ref, o_ref):
        ax = aggx_ref[...]
        ax = jnp.where(jnp.isfinite(ax), ax, 0.0)
        ar = aggrm_ref[...]
        ar = jnp.where(jnp.isfinite(ar), ar, 0.0)
        y = (jnp.dot(ax, Wla_ref[...], preferred_element_type=jnp.float32)
             + jnp.dot(ar, Wlb_ref[...], preferred_element_type=jnp.float32)
             + jnp.dot(x_ref[...], Wra_ref[...], preferred_element_type=jnp.float32)
             + jnp.dot(rm_ref[...], Wrb_ref[...], preferred_element_type=jnp.float32)
             + bls_ref[...])
        o_ref[...] = _bn_relu(y, gs_ref[...], bes_ref[...], True)

    return pl.pallas_call(
        body, out_shape=jax.ShapeDtypeStruct((NPAD, 128), jnp.float32),
    )(aggx, aggrm, x, rm16, Wls_a, Wls_b, Wrs_a, Wrs_b, bls, gs, bes)


def _final_stage(aggxs, xs, rm16, Wlr, blr, Wrr, gr, ber, Wr3p, br3,
                 Wlm, blm, Wrm, gm, bem, Wm3p, bm3):
    def body(aggxs_ref, xs_ref, rm_ref, Wlr_ref, blr_ref, Wrr_ref, gr_ref,
             ber_ref, Wr3_ref, br3_ref, Wlm_ref, blm_ref, Wrm_ref, gm_ref,
             bem_ref, Wm3_ref, bm3_ref, lsm_ref, o_ref):
        a = aggxs_ref[...]
        a = jnp.where(jnp.isfinite(a), a, 0.0)
        xs = xs_ref[...]
        rm = rm_ref[...]

        cols = lax.broadcasted_iota(jnp.int32, (NPAD, 16), 1)
        cmask = cols < 3
        mx = jnp.max(jnp.where(cmask, rm, -jnp.inf), axis=1, keepdims=True)
        se = jnp.sum(jnp.where(cmask, jnp.exp(rm - mx), 0.0), axis=1,
                     keepdims=True)
        lsm_ref[...] = rm - (mx + jnp.log(se))
        a0 = rm[:, 0:1]
        a1 = rm[:, 1:2]
        a2 = rm[:, 2:3]
        labels = jnp.where((a2 > a0) & (a2 > a1), 1.0, 0.0)

        yr = (jnp.dot(a, Wlr_ref[...], preferred_element_type=jnp.float32)
              + jnp.dot(xs, Wrr_ref[...], preferred_element_type=jnp.float32)
              + blr_ref[...])
        hr = _bn_relu(yr, gr_ref[...], ber_ref[...], True)
        rt = (jnp.dot(hr, Wr3_ref[...], preferred_element_type=jnp.float32)
              + br3_ref[...])

        ym = (jnp.dot(a, Wlm_ref[...], preferred_element_type=jnp.float32)
              + jnp.dot(xs, Wrm_ref[...], preferred_element_type=jnp.float32)
              + blm_ref[...])
        hm = _bn_relu(ym, gm_ref[...], bem_ref[...], True)
        md = (jnp.dot(hm, Wm3_ref[...], preferred_element_type=jnp.float32)
              + bm3_ref[...])

        ocols = lax.broadcasted_iota(jnp.int32, (NPAD, 128), 1)
        o_ref[...] = (jnp.where(ocols == 0, rt, 0.0)
                      + jnp.where(ocols == 1, md, 0.0)) * labels

    return pl.pallas_call(
        body,
        out_shape=(
            jax.ShapeDtypeStruct((NPAD, 16), jnp.float32),
            jax.ShapeDtypeStruct((NPAD, 128), jnp.float32),
        ),
    )(aggxs, xs, rm16, Wlr, blr, Wrr, gr, ber, Wr3p, br3, Wlm, blm, Wrm, gm,
      bem, Wm3p, bm3)


def kernel(x, edge_index, Wl1, bl1, Wr1, g1, be1, Wl2, bl2, Wr2, g2, be2,
           Wl4, bl4, Wr4, Wls, bls, Wrs, gs, bes, Wlr, blr, Wrr, gr, ber,
           Wr3, br3, Wlm, blm, Wrm, gm, bem, Wm3, bm3):
    src = edge_index[0]
    dst = edge_index[1]

    x_pad = jnp.zeros((NPAD, 128), jnp.float32).at[:N].set(x)

    Wls_a, Wls_b3 = Wls[:128], Wls[128:]
    Wrs_a, Wrs_b3 = Wrs[:128], Wrs[128:]
    Wls_b = jnp.zeros((16, 128), jnp.float32).at[:3].set(Wls_b3)
    Wrs_b = jnp.zeros((16, 128), jnp.float32).at[:3].set(Wrs_b3)
    Wl4p = jnp.pad(Wl4, ((0, 0), (0, 13)))
    Wr4p = jnp.pad(Wr4, ((0, 0), (0, 13)))
    bl4p = jnp.pad(bl4, (0, 13))
    Wr3p = jnp.pad(Wr3, ((0, 0), (0, 127)))
    Wm3p = jnp.pad(Wm3, ((0, 0), (1, 126)))

    srcl, ldl, cnts = _bin_edges(src, dst)

    aggx = _agg128(x_pad, srcl, ldl, cnts)
    h1 = _dense_stage(aggx, x_pad, Wl1, bl1, Wr1, g1, be1)
    aggh1 = _agg128(h1, srcl, ldl, cnts)
    h2 = _dense_stage(aggh1, h1, Wl2, bl2, Wr2, g2, be2)
    aggh2 = _agg64(h2, srcl, ldl, cnts)
    rm16 = _dense_nobn(aggh2, h2, Wl4p, bl4p, Wr4p)
    aggrm = _agg16(rm16, srcl, ldl, cnts)
    xs = _shared_stage(aggx, aggrm, x_pad, rm16, Wls_a, Wls_b, Wrs_a, Wrs_b,
                       bls, gs, bes)
    aggxs = _agg128(xs, srcl, ldl, cnts)
    lsm, out2 = _final_stage(aggxs, xs, rm16, Wlr, blr, Wrr, gr, ber, Wr3p,
                             br3, Wlm, blm, Wrm, gm, bem, Wm3p, bm3)

    rm_out = lsm[:N, :3]
    rt = out2[:N, 0]
    md = out2[:N, 1]
    return (rm_out, rt, md)


def _dense_nobn(agg, h, Wl, bl, Wr):
    H = Wl.shape[1]

    def body(agg_ref, h_ref, Wl_ref, bl_ref, Wr_ref, o_ref):
        a = agg_ref[...]
        a = jnp.where(jnp.isfinite(a), a, 0.0)
        o_ref[...] = (jnp.dot(a, Wl_ref[...], preferred_element_type=jnp.float32)
                      + jnp.dot(h_ref[...], Wr_ref[...],
                                preferred_element_type=jnp.float32)
                      + bl_ref[...])

    return pl.pallas_call(
        body, out_shape=jax.ShapeDtypeStruct((NPAD, H), jnp.float32),
    )(agg, h, Wl, bl, Wr)
